# baseline (speedup 1.0000x reference)
import functools
import math

import jax
import jax.numpy as jnp
from jax import lax
from jax.experimental import pallas as pl
from jax.experimental.pallas import tpu as pltpu
from jax.experimental.pallas import tpu_sc as plsc

S = 200
B = 1024
D = 100
PE_W = 128
CHUNK = 128
NCHUNKS = S * B // CHUNK
NWORKERS = 32
PER_W = NCHUNKS // NWORKERS
NPAIRS = PER_W // 2
CHUNKS_PER_S = B // CHUNK
IDX_PER_W = PER_W * CHUNK


TBLK = 2048


def _tr_kernel(x_ref, o_ref):
    r = lax.broadcasted_iota(jnp.int32, (D, D), 0)
    c = lax.broadcasted_iota(jnp.int32, (D, D), 1)
    eye = (r == c).astype(jnp.float32)
    o_ref[...] = lax.dot_general(
        x_ref[...], eye, (((0,), (0,)), ((), ())),
        preferred_element_type=jnp.float32)


def _transpose_table(table_t):
    vocab = table_t.shape[1]
    return pl.pallas_call(
        _tr_kernel,
        grid=((vocab + TBLK - 1) // TBLK,),
        in_specs=[pl.BlockSpec((D, TBLK), lambda i: (0, i))],
        out_specs=pl.BlockSpec((TBLK, D), lambda i: (i, 0)),
        out_shape=jax.ShapeDtypeStruct((vocab, D), jnp.float32),
    )(table_t)


def _pe_kernel(out_ref):
    s, w = out_ref.shape
    pos = lax.broadcasted_iota(jnp.int32, (s, w), 0).astype(jnp.float32)
    j = lax.broadcasted_iota(jnp.int32, (s, w), 1)
    d = jnp.where(j < D, j, jnp.where(j >= 124, j - 28, 0))
    valid = (j < D) | (j >= 124)
    df = d.astype(jnp.float32)
    i = jnp.floor(df * 0.5) * 2.0
    div = jnp.exp((i / float(D)) * math.log(10000.0))
    p = pos / div
    val = jnp.where(d % 2 == 0, jnp.sin(p), jnp.cos(p))
    out_ref[...] = jnp.where(valid, val, 0.0)


def _make_pe():
    return pl.pallas_call(
        _pe_kernel,
        out_shape=jax.ShapeDtypeStruct((S, PE_W), jnp.float32),
    )()


@functools.partial(
    pl.kernel,
    out_type=jax.ShapeDtypeStruct((S * B, D), jnp.float32),
    mesh=plsc.VectorSubcoreMesh(core_axis_name="c", subcore_axis_name="s"),
    scratch_types=[
        pltpu.VMEM((IDX_PER_W,), jnp.int32),
        pltpu.VMEM((CHUNK, D), jnp.float32),
        pltpu.VMEM((CHUNK, D), jnp.float32),
        pltpu.VMEM((S, PE_W), jnp.float32),
        pltpu.SemaphoreType.DMA,
        pltpu.SemaphoreType.DMA,
        pltpu.SemaphoreType.DMA,
        pltpu.SemaphoreType.DMA,
    ],
)
def _gather_pe(x_hbm, table_hbm, pe_hbm, out_hbm,
               idx_all, buf_a, buf_b, pe_all,
               sem_a, sem_b, osem_a, osem_b):
    wid = lax.axis_index("s") * 2 + lax.axis_index("c")
    c0 = wid * PER_W

    pltpu.sync_copy(x_hbm.at[wid], idx_all)
    pltpu.sync_copy(pe_hbm, pe_all)

    def fire(t, buf, sem):
        def fj(j, fc):
            vec = idx_all[pl.ds(t * CHUNK + j * 16, 16)]
            for k in range(16):
                pltpu.async_copy(
                    table_hbm.at[pl.ds(vec[k], 1)],
                    buf.at[pl.ds(j * 16 + k, 1)],
                    sem,
                )
            return fc

        lax.fori_loop(0, CHUNK // 16, fj, 0)

    def drain(buf, sem):
        pltpu.make_async_copy(table_hbm.at[pl.ds(0, CHUNK)], buf, sem).wait()

    def adds(buf, t):
        srel = (c0 + t) // CHUNKS_PER_S
        pes = [pe_all[srel, pl.ds(k * 16, 16)] for k in range(6)]
        pe_tail = pe_all[srel, pl.ds(112, 16)]

        def row_body(h, rc):
            for r in (2 * h, 2 * h + 1):
                for k in range(6):
                    buf[r, pl.ds(k * 16, 16)] = (
                        buf[r, pl.ds(k * 16, 16)] + pes[k]
                    )
                buf[r, pl.ds(84, 16)] = buf[r, pl.ds(84, 16)] + pe_tail
            return rc

        lax.fori_loop(0, CHUNK // 2, row_body, 0)

    def out_copy(t, buf, osem):
        pltpu.async_copy(
            buf, out_hbm.at[pl.ds((c0 + t) * CHUNK, CHUNK)], osem
        )

    fire(0, buf_a, sem_a)

    def pair_body(p, carry):
        t0 = 2 * p

        @pl.when(p > 0)
        def _():
            drain(buf_b, osem_b)

        fire(t0 + 1, buf_b, sem_b)
        drain(buf_a, sem_a)
        adds(buf_a, t0)
        out_copy(t0, buf_a, osem_a)

        @pl.when(p < NPAIRS - 1)
        def _():
            drain(buf_a, osem_a)
            fire(t0 + 2, buf_a, sem_a)

        drain(buf_b, sem_b)
        adds(buf_b, t0 + 1)
        out_copy(t0 + 1, buf_b, osem_b)
        return carry

    lax.fori_loop(0, NPAIRS, pair_body, 0)
    drain(buf_a, osem_a)
    drain(buf_b, osem_b)


def kernel(x, table):
    pe = _make_pe()
    table_rm = _transpose_table(table.T)
    x_flat = x.reshape(NWORKERS, IDX_PER_W).astype(jnp.int32)
    out = _gather_pe(x_flat, table_rm, pe)
    return out.reshape(S, B, D)


# speedup vs baseline: 1.2069x; 1.2069x over previous
import functools
import math

import jax
import jax.numpy as jnp
from jax import lax
from jax.experimental import pallas as pl
from jax.experimental.pallas import tpu as pltpu
from jax.experimental.pallas import tpu_sc as plsc

S = 200
B = 1024
D = 100
PE_W = 128
CHUNK = 128
NCHUNKS = S * B // CHUNK
NWORKERS = 32
PER_W = NCHUNKS // NWORKERS
NPAIRS = PER_W // 2
CHUNKS_PER_S = B // CHUNK
IDX_PER_W = PER_W * CHUNK


def _pe_kernel(out_ref):
    s, w = out_ref.shape
    pos = lax.broadcasted_iota(jnp.int32, (s, w), 0).astype(jnp.float32)
    j = lax.broadcasted_iota(jnp.int32, (s, w), 1)
    d = jnp.where(j < D, j, jnp.where(j >= 124, j - 28, 0))
    valid = (j < D) | (j >= 124)
    df = d.astype(jnp.float32)
    i = jnp.floor(df * 0.5) * 2.0
    div = jnp.exp((i / float(D)) * math.log(10000.0))
    p = pos / div
    val = jnp.where(d % 2 == 0, jnp.sin(p), jnp.cos(p))
    out_ref[...] = jnp.where(valid, val, 0.0)


def _make_pe():
    return pl.pallas_call(
        _pe_kernel,
        out_shape=jax.ShapeDtypeStruct((S, PE_W), jnp.float32),
    )()


@functools.partial(
    pl.kernel,
    out_type=jax.ShapeDtypeStruct((S * B, D), jnp.float32),
    mesh=plsc.VectorSubcoreMesh(core_axis_name="c", subcore_axis_name="s"),
    scratch_types=[
        pltpu.VMEM((IDX_PER_W,), jnp.int32),
        pltpu.VMEM((CHUNK, D), jnp.float32),
        pltpu.VMEM((CHUNK, D), jnp.float32),
        pltpu.VMEM((S, PE_W), jnp.float32),
        pltpu.SemaphoreType.DMA,
        pltpu.SemaphoreType.DMA,
        pltpu.SemaphoreType.DMA,
        pltpu.SemaphoreType.DMA,
    ],
)
def _gather_pe(x_hbm, table_hbm, pe_hbm, out_hbm,
               idx_all, buf_a, buf_b, pe_all,
               sem_a, sem_b, osem_a, osem_b):
    wid = lax.axis_index("s") * 2 + lax.axis_index("c")
    c0 = wid * PER_W

    pltpu.sync_copy(x_hbm.at[wid], idx_all)
    pltpu.sync_copy(pe_hbm, pe_all)

    def fire(t, buf, sem):
        def fj(j, fc):
            vec = idx_all[pl.ds(t * CHUNK + j * 16, 16)]
            for k in range(16):
                pltpu.async_copy(
                    table_hbm.at[pl.ds(vec[k], 1)],
                    buf.at[pl.ds(j * 16 + k, 1)],
                    sem,
                )
            return fc

        lax.fori_loop(0, CHUNK // 16, fj, 0)

    def drain(buf, sem):
        pltpu.make_async_copy(table_hbm.at[pl.ds(0, CHUNK)], buf, sem).wait()

    def adds(buf, t):
        srel = (c0 + t) // CHUNKS_PER_S
        pes = [pe_all[srel, pl.ds(k * 16, 16)] for k in range(6)]
        pe_tail = pe_all[srel, pl.ds(112, 16)]

        def row_body(h, rc):
            for r in (2 * h, 2 * h + 1):
                for k in range(6):
                    buf[r, pl.ds(k * 16, 16)] = (
                        buf[r, pl.ds(k * 16, 16)] + pes[k]
                    )
                buf[r, pl.ds(84, 16)] = buf[r, pl.ds(84, 16)] + pe_tail
            return rc

        lax.fori_loop(0, CHUNK // 2, row_body, 0)

    def out_copy(t, buf, osem):
        pltpu.async_copy(
            buf, out_hbm.at[pl.ds((c0 + t) * CHUNK, CHUNK)], osem
        )

    fire(0, buf_a, sem_a)

    def pair_body(p, carry):
        t0 = 2 * p

        @pl.when(p > 0)
        def _():
            drain(buf_b, osem_b)

        fire(t0 + 1, buf_b, sem_b)
        drain(buf_a, sem_a)
        adds(buf_a, t0)
        out_copy(t0, buf_a, osem_a)

        @pl.when(p < NPAIRS - 1)
        def _():
            drain(buf_a, osem_a)
            fire(t0 + 2, buf_a, sem_a)

        drain(buf_b, sem_b)
        adds(buf_b, t0 + 1)
        out_copy(t0 + 1, buf_b, osem_b)
        return carry

    lax.fori_loop(0, NPAIRS, pair_body, 0)
    drain(buf_a, osem_a)
    drain(buf_b, osem_b)


def kernel(x, table):
    pe = _make_pe()
    x_flat = x.reshape(NWORKERS, IDX_PER_W).astype(jnp.int32)
    out = _gather_pe(x_flat, table, pe)
    return out.reshape(S, B, D)


# speedup vs baseline: 1.2093x; 1.0020x over previous
import functools
import math

import jax
import jax.numpy as jnp
from jax import lax
from jax.experimental import pallas as pl
from jax.experimental.pallas import tpu as pltpu
from jax.experimental.pallas import tpu_sc as plsc

S = 200
B = 1024
D = 100
PE_W = 128
CHUNK = 128
NCHUNKS = S * B // CHUNK
NWORKERS = 32
PER_W = NCHUNKS // NWORKERS
NTRIPLES = PER_W // 3
CHUNKS_PER_S = B // CHUNK
IDX_PER_W = PER_W * CHUNK


def _pe_kernel(out_ref):
    s, w = out_ref.shape
    pos = lax.broadcasted_iota(jnp.int32, (s, w), 0).astype(jnp.float32)
    j = lax.broadcasted_iota(jnp.int32, (s, w), 1)
    d = jnp.where(j < D, j, jnp.where(j >= 124, j - 28, 0))
    valid = (j < D) | (j >= 124)
    df = d.astype(jnp.float32)
    i = jnp.floor(df * 0.5) * 2.0
    div = jnp.exp((i / float(D)) * math.log(10000.0))
    p = pos / div
    val = jnp.where(d % 2 == 0, jnp.sin(p), jnp.cos(p))
    out_ref[...] = jnp.where(valid, val, 0.0)


def _make_pe():
    return pl.pallas_call(
        _pe_kernel,
        out_shape=jax.ShapeDtypeStruct((S, PE_W), jnp.float32),
    )()


@functools.partial(
    pl.kernel,
    out_type=jax.ShapeDtypeStruct((S * B, D), jnp.float32),
    mesh=plsc.VectorSubcoreMesh(core_axis_name="c", subcore_axis_name="s"),
    scratch_types=[
        pltpu.VMEM((IDX_PER_W,), jnp.int32),
        pltpu.VMEM((CHUNK, D), jnp.float32),
        pltpu.VMEM((CHUNK, D), jnp.float32),
        pltpu.VMEM((CHUNK, D), jnp.float32),
        pltpu.VMEM((S, PE_W), jnp.float32),
        pltpu.SemaphoreType.DMA,
        pltpu.SemaphoreType.DMA,
        pltpu.SemaphoreType.DMA,
        pltpu.SemaphoreType.DMA,
        pltpu.SemaphoreType.DMA,
        pltpu.SemaphoreType.DMA,
    ],
)
def _gather_pe(x_hbm, table_hbm, pe_hbm, out_hbm,
               idx_all, buf_a, buf_b, buf_c, pe_all,
               sem_a, sem_b, sem_c, osem_a, osem_b, osem_c):
    wid = lax.axis_index("s") * 2 + lax.axis_index("c")
    c0 = wid * PER_W

    pltpu.sync_copy(x_hbm.at[wid], idx_all)
    pltpu.sync_copy(pe_hbm, pe_all)

    def fire(t, buf, sem):
        def fj(j, fc):
            vec = idx_all[pl.ds(t * CHUNK + j * 16, 16)]
            for k in range(16):
                pltpu.async_copy(
                    table_hbm.at[pl.ds(vec[k], 1)],
                    buf.at[pl.ds(j * 16 + k, 1)],
                    sem,
                )
            return fc

        lax.fori_loop(0, CHUNK // 16, fj, 0)

    def drain(buf, sem):
        pltpu.make_async_copy(table_hbm.at[pl.ds(0, CHUNK)], buf, sem).wait()

    def adds(buf, t):
        srel = (c0 + t) // CHUNKS_PER_S
        pes = [pe_all[srel, pl.ds(k * 16, 16)] for k in range(6)]
        pe_tail = pe_all[srel, pl.ds(112, 16)]

        def row_body(h, rc):
            for r in (2 * h, 2 * h + 1):
                for k in range(6):
                    buf[r, pl.ds(k * 16, 16)] = (
                        buf[r, pl.ds(k * 16, 16)] + pes[k]
                    )
                buf[r, pl.ds(84, 16)] = buf[r, pl.ds(84, 16)] + pe_tail
            return rc

        lax.fori_loop(0, CHUNK // 2, row_body, 0)

    def out_copy(t, buf, osem):
        pltpu.async_copy(
            buf, out_hbm.at[pl.ds((c0 + t) * CHUNK, CHUNK)], osem
        )

    def process(t, buf, sem, osem):
        drain(buf, sem)
        adds(buf, t)
        out_copy(t, buf, osem)

    fire(0, buf_a, sem_a)
    fire(1, buf_b, sem_b)

    def triple_body(q, carry):
        t0 = 3 * q

        @pl.when(q > 0)
        def _():
            drain(buf_c, osem_c)

        fire(t0 + 2, buf_c, sem_c)
        process(t0, buf_a, sem_a, osem_a)

        @pl.when(q < NTRIPLES - 1)
        def _():
            drain(buf_a, osem_a)
            fire(t0 + 3, buf_a, sem_a)

        process(t0 + 1, buf_b, sem_b, osem_b)

        @pl.when(q < NTRIPLES - 1)
        def _():
            drain(buf_b, osem_b)
            fire(t0 + 4, buf_b, sem_b)

        process(t0 + 2, buf_c, sem_c, osem_c)
        return carry

    lax.fori_loop(0, NTRIPLES, triple_body, 0)

    drain(buf_a, osem_a)
    fire(PER_W - 2, buf_a, sem_a)
    drain(buf_b, osem_b)
    fire(PER_W - 1, buf_b, sem_b)
    process(PER_W - 2, buf_a, sem_a, osem_a)
    process(PER_W - 1, buf_b, sem_b, osem_b)
    drain(buf_a, osem_a)
    drain(buf_b, osem_b)
    drain(buf_c, osem_c)


def kernel(x, table):
    pe = _make_pe()
    x_flat = x.reshape(NWORKERS, IDX_PER_W).astype(jnp.int32)
    out = _gather_pe(x_flat, table, pe)
    return out.reshape(S, B, D)
